# Initial kernel scaffold; baseline (speedup 1.0000x reference)
import jax
import jax.numpy as jnp
from jax.experimental import pallas as pl


def kernel(input, table):
    raise NotImplementedError("write your pallas kernel here")


# speedup vs baseline: 1.5312x; 1.5312x over previous
import functools

import jax
import jax.numpy as jnp
from jax import lax
from jax.experimental import pallas as pl
from jax.experimental.pallas import tpu as pltpu
from jax.experimental.pallas import tpu_sc as plsc

B, S = 4096, 50
D = 300
ROWS = B * S

NC, NS = 2, 16
NW = NC * NS
PER_W = ROWS // NW
CHUNK = 128
NCHUNK = PER_W // CHUNK
GRP = 16

_mesh = plsc.VectorSubcoreMesh(core_axis_name="c", subcore_axis_name="s")


@functools.partial(
    pl.kernel,
    mesh=_mesh,
    out_type=jax.ShapeDtypeStruct((ROWS, D), jnp.float32),
    scratch_types=[
        pltpu.VMEM((PER_W,), jnp.int32),
        pltpu.VMEM((CHUNK, D), jnp.float32),
        pltpu.VMEM((CHUNK, D), jnp.float32),
        pltpu.SemaphoreType.DMA,
        pltpu.SemaphoreType.DMA,
        pltpu.SemaphoreType.DMA,
        pltpu.SemaphoreType.DMA,
    ],
)
def _emb_gather(idx_hbm, table_hbm, out_hbm, idx_v, buf0, buf1,
                gsem0, gsem1, wsem0, wsem1):
    wid = lax.axis_index("s") * NC + lax.axis_index("c")
    base = wid * PER_W
    pltpu.sync_copy(idx_hbm.at[pl.ds(base, PER_W)], idx_v)

    bufs = (buf0, buf1)
    gsems = (gsem0, gsem1)
    wsems = (wsem0, wsem1)

    def fire_gathers(c, buf, gsem):
        def grp_body(g, carry):
            v = idx_v[pl.ds(c * CHUNK + g * GRP, GRP)]
            for j in range(GRP):
                i = v[j]
                pltpu.async_copy(table_hbm.at[pl.ds(i, 1)],
                                 buf.at[pl.ds(g * GRP + j, 1)], gsem)
            return carry
        lax.fori_loop(0, CHUNK // GRP, grp_body, 0)

    def drain_gathers(buf, gsem):
        pltpu.make_async_copy(table_hbm.at[pl.ds(0, CHUNK)], buf, gsem).wait()

    def fire_write(c, buf, wsem):
        pltpu.async_copy(buf, out_hbm.at[pl.ds(base + c * CHUNK, CHUNK)], wsem)

    def drain_write(buf, wsem):
        pltpu.make_async_copy(buf, out_hbm.at[pl.ds(base, CHUNK)], wsem).wait()

    def stage(c_prev, c_next, pa, pb):
        @pl.when(c_next >= 2)
        def _():
            drain_write(bufs[pb], wsems[pb])
        fire_gathers(c_next, bufs[pb], gsems[pb])
        drain_gathers(bufs[pa], gsems[pa])
        fire_write(c_prev, bufs[pa], wsems[pa])

    fire_gathers(0, buf0, gsem0)

    def main_body(g, carry):
        stage(2 * g, 2 * g + 1, 0, 1)
        stage(2 * g + 1, 2 * g + 2, 1, 0)
        return carry

    lax.fori_loop(0, (NCHUNK - 2) // 2, main_body, 0)

    stage(NCHUNK - 2, NCHUNK - 1, 0, 1)
    drain_gathers(buf1, gsem1)
    fire_write(NCHUNK - 1, buf1, wsem1)
    drain_write(buf0, wsem0)
    drain_write(buf1, wsem1)


def kernel(input, table):
    idx = input.astype(jnp.int32).reshape(ROWS)
    out = _emb_gather(idx, table)
    return out.reshape(B, S, D)
